# baseline (speedup 1.0000x reference)
import jax
import jax.numpy as jnp
from jax.experimental import pallas as pl
from jax.experimental.pallas import tpu as pltpu

CHUNK = 64
T_BLOCK = 1024


def _body(x_ref, o_ref):
    x = x_ref[...]
    pos = jax.lax.broadcasted_iota(jnp.int32, x.shape, 1) % CHUNK
    v = x
    for k in (1, 2, 4, 8, 16, 32):
        v = v + jnp.where(pos >= k, pltpu.roll(v, k, axis=1), 0.0)
    o_ref[...] = v


def kernel(g):
    B, T, H = g.shape
    return pl.pallas_call(
        _body,
        out_shape=jax.ShapeDtypeStruct(g.shape, jnp.float32),
        grid=(T // T_BLOCK,),
        in_specs=[pl.BlockSpec((B, T_BLOCK, H), lambda i: (0, i, 0))],
        out_specs=pl.BlockSpec((B, T_BLOCK, H), lambda i: (0, i, 0)),
    )(g)


# speedup vs baseline: 3.2021x; 3.2021x over previous
import jax
import jax.numpy as jnp
from jax.experimental import pallas as pl
from jax.experimental.pallas import tpu as pltpu

CHUNK = 64
T_BLOCK = 1024


def _body(x_ref, o_ref):
    x = x_ref[...]
    pos = jax.lax.broadcasted_iota(jnp.int32, x.shape, 2) % CHUNK
    v = x
    for k in (1, 2, 4, 8, 16, 32):
        v = v + jnp.where(pos >= k, pltpu.roll(v, k, axis=2), 0.0)
    o_ref[...] = v


def kernel(g):
    B, T, H = g.shape
    gt = jnp.transpose(g, (0, 2, 1))
    ot = pl.pallas_call(
        _body,
        out_shape=jax.ShapeDtypeStruct((B, H, T), jnp.float32),
        grid=(T // T_BLOCK,),
        in_specs=[pl.BlockSpec((B, H, T_BLOCK), lambda i: (0, 0, i))],
        out_specs=pl.BlockSpec((B, H, T_BLOCK), lambda i: (0, 0, i)),
    )(gt)
    return jnp.transpose(ot, (0, 2, 1))
